# Initial kernel scaffold; baseline (speedup 1.0000x reference)
import jax
import jax.numpy as jnp
from jax.experimental import pallas as pl


def kernel(x, fixed_knn_graph, W1, g1, b1, W2, g2, b2):
    raise NotImplementedError("write your pallas kernel here")


# speedup vs baseline: 14.2993x; 14.2993x over previous
import jax
import jax.numpy as jnp
from jax import lax
from jax.experimental import pallas as pl
from jax.experimental.pallas import tpu as pltpu
from jax.experimental.pallas import tpu_sc as plsc

_EPS = 1e-5
_SLOPE = 0.2


def _tables(xt, wnT, wcT):
    BN, F = xt.shape
    CN = 2048
    O = wnT.shape[1]

    def body(x_ref, wn_ref, wc_ref, pt_ref, ut_ref):
        xb = x_ref[...]
        wn = wn_ref[...]
        pt_ref[...] = jnp.dot(xb, wn, preferred_element_type=jnp.float32)
        ut_ref[...] = jnp.dot(xb, wc_ref[...] - wn,
                              preferred_element_type=jnp.float32)

    return pl.pallas_call(
        body,
        grid=(BN // CN,),
        in_specs=[
            pl.BlockSpec((CN, F), lambda t: (t, 0)),
            pl.BlockSpec((F, O), lambda t: (0, 0)),
            pl.BlockSpec((F, O), lambda t: (0, 0)),
        ],
        out_specs=[
            pl.BlockSpec((CN, O), lambda t: (t, 0)),
            pl.BlockSpec((CN, O), lambda t: (t, 0)),
        ],
        out_shape=[jax.ShapeDtypeStruct((BN, O), jnp.float32)] * 2,
    )(xt, wnT, wcT)


def _sc_gather(pt, idx2d):
    NB, LW = idx2d.shape
    O = pt.shape[1]
    E = NB * LW
    info = plsc.get_sparse_core_info()
    nc = info.num_cores
    NW = nc * info.num_subcores
    GG = 4
    blocks_per_w = NB // NW
    outer = blocks_per_w // GG

    def body(pt_hbm, idx_hbm, g_hbm, idx_v, rows_v, sem):
        wid = lax.axis_index("s") * nc + lax.axis_index("c")
        blk0 = wid * blocks_per_w

        def step(i, carry):
            b = blk0 + i * GG
            pltpu.sync_copy(idx_hbm.at[pl.ds(b, GG)], idx_v)
            cps = [
                pltpu.async_copy(pt_hbm.at[idx_v.at[j]],
                                 rows_v.at[pl.ds(j * LW, LW)], sem)
                for j in range(GG)
            ]
            for c in cps:
                c.wait()
            pltpu.sync_copy(rows_v, g_hbm.at[pl.ds(b * LW, GG * LW)])
            return carry

        lax.fori_loop(0, outer, step, 0)

    return pl.kernel(
        body,
        out_type=jax.ShapeDtypeStruct((E, O), jnp.float32),
        mesh=plsc.VectorSubcoreMesh(core_axis_name="c", subcore_axis_name="s"),
        compiler_params=pltpu.CompilerParams(use_tc_tiling_on_sc=False),
        scratch_types=[
            pltpu.VMEM((GG, LW), jnp.int32),
            pltpu.VMEM((GG * LW, O), jnp.float32),
            pltpu.SemaphoreType.DMA,
        ],
    )(pt, idx2d)


def _stats1(gk, ut):
    K, BN, O = gk.shape
    PTS = 256
    T = BN // PTS

    def body(g_ref, u_ref, s_ref):
        t = pl.program_id(0)
        u = u_ref[...]
        ps = jnp.zeros((1, O), jnp.float32)
        pss = jnp.zeros((1, O), jnp.float32)
        for k in range(K):
            z = g_ref[k] + u
            ps = ps + jnp.sum(z, axis=0, keepdims=True)
            pss = pss + jnp.sum(z * z, axis=0, keepdims=True)

        @pl.when(t == 0)
        def _():
            s_ref[...] = jnp.zeros_like(s_ref)

        s_ref[0:1, :] += ps
        s_ref[1:2, :] += pss

    return pl.pallas_call(
        body,
        grid=(T,),
        in_specs=[
            pl.BlockSpec((K, PTS, O), lambda t: (0, t, 0)),
            pl.BlockSpec((PTS, O), lambda t: (t, 0)),
        ],
        out_specs=pl.BlockSpec((2, O), lambda t: (0, 0)),
        out_shape=jax.ShapeDtypeStruct((2, O), jnp.float32),
    )(gk, ut)


def _main_pass(gk, ut, st1, gb1, w2t):
    K, BN, O = gk.shape
    PTS = 256
    T = BN // PTS
    M = float(BN * K)

    def body(g_ref, u_ref, st1_ref, gb1_ref, w2t_ref, m_ref, s2_ref):
        t = pl.program_id(0)
        mean1 = st1_ref[0:1, :] / M
        ex2 = st1_ref[1:2, :] / M
        var1 = ex2 - mean1 * mean1
        s1 = gb1_ref[0:1, :] * lax.rsqrt(var1 + _EPS)
        t1 = gb1_ref[1:2, :] - mean1 * s1
        u = u_ref[...]
        w2 = w2t_ref[...]
        ps = jnp.zeros((1, O), jnp.float32)
        pss = jnp.zeros((1, O), jnp.float32)
        mx = None
        for k in range(K):
            z = (g_ref[k] + u) * s1 + t1
            a = jnp.where(z >= 0, z, _SLOPE * z)
            h2 = jnp.dot(a, w2, preferred_element_type=jnp.float32)
            ps = ps + jnp.sum(h2, axis=0, keepdims=True)
            pss = pss + jnp.sum(h2 * h2, axis=0, keepdims=True)
            mx = h2 if mx is None else jnp.maximum(mx, h2)
        m_ref[...] = mx

        @pl.when(t == 0)
        def _():
            s2_ref[...] = jnp.zeros_like(s2_ref)

        s2_ref[0:1, :] += ps
        s2_ref[1:2, :] += pss

    return pl.pallas_call(
        body,
        grid=(T,),
        in_specs=[
            pl.BlockSpec((K, PTS, O), lambda t: (0, t, 0)),
            pl.BlockSpec((PTS, O), lambda t: (t, 0)),
            pl.BlockSpec((2, O), lambda t: (0, 0)),
            pl.BlockSpec((2, O), lambda t: (0, 0)),
            pl.BlockSpec((O, O), lambda t: (0, 0)),
        ],
        out_specs=[
            pl.BlockSpec((PTS, O), lambda t: (t, 0)),
            pl.BlockSpec((2, O), lambda t: (0, 0)),
        ],
        out_shape=[
            jax.ShapeDtypeStruct((BN, O), jnp.float32),
            jax.ShapeDtypeStruct((2, O), jnp.float32),
        ],
    )(gk, ut, st1, gb1, w2t)


def _finalize(m, st2, gb2, M):
    BN, O = m.shape
    PTS = 2048
    Mf = float(M)

    def body(m_ref, st2_ref, gb2_ref, o_ref):
        mean2 = st2_ref[0:1, :] / Mf
        var2 = st2_ref[1:2, :] / Mf - mean2 * mean2
        s2 = gb2_ref[0:1, :] * lax.rsqrt(var2 + _EPS)
        t2 = gb2_ref[1:2, :] - mean2 * s2
        z = m_ref[...] * s2 + t2
        o_ref[...] = jnp.where(z >= 0, z, _SLOPE * z)

    return pl.pallas_call(
        body,
        grid=(BN // PTS,),
        in_specs=[
            pl.BlockSpec((PTS, O), lambda t: (t, 0)),
            pl.BlockSpec((2, O), lambda t: (0, 0)),
            pl.BlockSpec((2, O), lambda t: (0, 0)),
        ],
        out_specs=pl.BlockSpec((PTS, O), lambda t: (t, 0)),
        out_shape=jax.ShapeDtypeStruct((BN, O), jnp.float32),
    )(m, st2, gb2)


def kernel(x, fixed_knn_graph, W1, g1, b1, W2, g2, b2):
    B, F, N = x.shape
    K = fixed_knn_graph.shape[-1]
    BN = B * N
    E = BN * K

    xt = jnp.transpose(x, (0, 2, 1)).reshape(BN, F)
    wnT = jnp.transpose(W1[:, :F])
    wcT = jnp.transpose(W1[:, F:])
    pt, ut = _tables(xt, wnT, wcT)

    idx = fixed_knn_graph.astype(jnp.int32)
    idx = idx + (jnp.arange(B, dtype=jnp.int32) * N)[:, None, None]
    idx_kmaj = jnp.transpose(idx, (2, 0, 1)).reshape(E // 128, 128)

    g = _sc_gather(pt, idx_kmaj)
    gk = g.reshape(K, BN, F)

    st1 = _stats1(gk, ut)
    gb1 = jnp.stack([g1, b1])
    gb2 = jnp.stack([g2, b2])
    m, st2 = _main_pass(gk, ut, st1, gb1, jnp.transpose(W2))
    y = _finalize(m, st2, gb2, E)
    return y.reshape(B, N, -1).transpose(0, 2, 1)


# speedup vs baseline: 17.1160x; 1.1970x over previous
import jax
import jax.numpy as jnp
from jax import lax
from jax.experimental import pallas as pl
from jax.experimental.pallas import tpu as pltpu
from jax.experimental.pallas import tpu_sc as plsc

_EPS = 1e-5
_SLOPE = 0.2


def _tables(xt, wnT, wcT):
    BN, F = xt.shape
    CN = 2048
    O = wnT.shape[1]

    def body(x_ref, wn_ref, wc_ref, pt_ref, ut_ref):
        xb = x_ref[...]
        wn = wn_ref[...]
        pt_ref[...] = jnp.dot(
            xb, wn, preferred_element_type=jnp.float32).astype(jnp.bfloat16)
        ut_ref[...] = jnp.dot(xb, wc_ref[...] - wn,
                              preferred_element_type=jnp.float32)

    return pl.pallas_call(
        body,
        grid=(BN // CN,),
        in_specs=[
            pl.BlockSpec((CN, F), lambda t: (t, 0)),
            pl.BlockSpec((F, O), lambda t: (0, 0)),
            pl.BlockSpec((F, O), lambda t: (0, 0)),
        ],
        out_specs=[
            pl.BlockSpec((CN, O), lambda t: (t, 0)),
            pl.BlockSpec((CN, O), lambda t: (t, 0)),
        ],
        out_shape=[
            jax.ShapeDtypeStruct((BN, O), jnp.bfloat16),
            jax.ShapeDtypeStruct((BN, O), jnp.float32),
        ],
    )(xt, wnT, wcT)


def _sc_gather(pt, idx2d):
    NB, LW = idx2d.shape
    O = pt.shape[1]
    E = NB * LW
    info = plsc.get_sparse_core_info()
    nc = info.num_cores
    NW = nc * info.num_subcores
    GG = 4
    blocks_per_w = NB // NW
    outer = blocks_per_w // GG

    def body(pt_hbm, idx_hbm, g_hbm, idx_v, rows_v, sem):
        wid = lax.axis_index("s") * nc + lax.axis_index("c")
        blk0 = wid * blocks_per_w

        def step(i, carry):
            b = blk0 + i * GG
            pltpu.sync_copy(idx_hbm.at[pl.ds(b, GG)], idx_v)
            cps = [
                pltpu.async_copy(pt_hbm.at[idx_v.at[j]],
                                 rows_v.at[pl.ds(j * LW, LW)], sem)
                for j in range(GG)
            ]
            for c in cps:
                c.wait()
            pltpu.sync_copy(rows_v, g_hbm.at[pl.ds(b * LW, GG * LW)])
            return carry

        lax.fori_loop(0, outer, step, 0)

    return pl.kernel(
        body,
        out_type=jax.ShapeDtypeStruct((E, O), pt.dtype),
        mesh=plsc.VectorSubcoreMesh(core_axis_name="c", subcore_axis_name="s"),
        compiler_params=pltpu.CompilerParams(use_tc_tiling_on_sc=False),
        scratch_types=[
            pltpu.VMEM((GG, LW), jnp.int32),
            pltpu.VMEM((GG * LW, O), pt.dtype),
            pltpu.SemaphoreType.DMA,
        ],
    )(pt, idx2d)


def _stats1(gp, utp):
    K, R, C2 = gp.shape
    RT = 256
    T = R // RT

    def body(g_ref, u_ref, s_ref):
        t = pl.program_id(0)
        u = u_ref[...]
        ps = jnp.zeros((1, C2), jnp.float32)
        pss = jnp.zeros((1, C2), jnp.float32)
        for k in range(K):
            z = g_ref[k].astype(jnp.float32) + u
            ps = ps + jnp.sum(z, axis=0, keepdims=True)
            pss = pss + jnp.sum(z * z, axis=0, keepdims=True)

        @pl.when(t == 0)
        def _():
            s_ref[...] = jnp.zeros_like(s_ref)

        s_ref[0:1, :] += ps
        s_ref[1:2, :] += pss

    return pl.pallas_call(
        body,
        grid=(T,),
        in_specs=[
            pl.BlockSpec((K, RT, C2), lambda t: (0, t, 0)),
            pl.BlockSpec((RT, C2), lambda t: (t, 0)),
        ],
        out_specs=pl.BlockSpec((2, C2), lambda t: (0, 0)),
        out_shape=jax.ShapeDtypeStruct((2, C2), jnp.float32),
    )(gp, utp)


def _affine_from_stats(st, gb, m_count):
    c = st.shape[1] // 2
    mean = (st[0:1, :c] + st[0:1, c:]) / m_count
    ex2 = (st[1:2, :c] + st[1:2, c:]) / m_count
    var = ex2 - mean * mean
    s = gb[0:1, :] * lax.rsqrt(var + _EPS)
    t = gb[1:2, :] - mean * s
    return (jnp.concatenate([s, s], axis=1),
            jnp.concatenate([t, t], axis=1))


def _main_pass(gp, utp, st1, gb1, w2d, M):
    K, R, C2 = gp.shape
    RT = 256
    T = R // RT
    Mf = float(M)

    def body(g_ref, u_ref, st1_ref, gb1_ref, w2d_ref, m_ref, s2_ref):
        t = pl.program_id(0)
        s1, t1 = _affine_from_stats(st1_ref[...], gb1_ref[...], Mf)
        u = u_ref[...]
        w2 = w2d_ref[...]
        ps = jnp.zeros((1, C2), jnp.float32)
        pss = jnp.zeros((1, C2), jnp.float32)
        mx = None
        for k in range(K):
            z = (g_ref[k].astype(jnp.float32) + u) * s1 + t1
            a = jnp.where(z >= 0, z, _SLOPE * z)
            h2 = jnp.dot(a, w2, preferred_element_type=jnp.float32)
            ps = ps + jnp.sum(h2, axis=0, keepdims=True)
            pss = pss + jnp.sum(h2 * h2, axis=0, keepdims=True)
            mx = h2 if mx is None else jnp.maximum(mx, h2)
        m_ref[...] = mx

        @pl.when(t == 0)
        def _():
            s2_ref[...] = jnp.zeros_like(s2_ref)

        s2_ref[0:1, :] += ps
        s2_ref[1:2, :] += pss

    return pl.pallas_call(
        body,
        grid=(T,),
        in_specs=[
            pl.BlockSpec((K, RT, C2), lambda t: (0, t, 0)),
            pl.BlockSpec((RT, C2), lambda t: (t, 0)),
            pl.BlockSpec((2, C2), lambda t: (0, 0)),
            pl.BlockSpec((2, C2 // 2), lambda t: (0, 0)),
            pl.BlockSpec((C2, C2), lambda t: (0, 0)),
        ],
        out_specs=[
            pl.BlockSpec((RT, C2), lambda t: (t, 0)),
            pl.BlockSpec((2, C2), lambda t: (0, 0)),
        ],
        out_shape=[
            jax.ShapeDtypeStruct((R, C2), jnp.float32),
            jax.ShapeDtypeStruct((2, C2), jnp.float32),
        ],
    )(gp, utp, st1, gb1, w2d)


def _finalize(m, st2, gb2, M):
    R, C2 = m.shape
    RT = 2048
    Mf = float(M)

    def body(m_ref, st2_ref, gb2_ref, o_ref):
        s2, t2 = _affine_from_stats(st2_ref[...], gb2_ref[...], Mf)
        z = m_ref[...] * s2 + t2
        o_ref[...] = jnp.where(z >= 0, z, _SLOPE * z)

    return pl.pallas_call(
        body,
        grid=(R // RT,),
        in_specs=[
            pl.BlockSpec((RT, C2), lambda t: (t, 0)),
            pl.BlockSpec((2, C2), lambda t: (0, 0)),
            pl.BlockSpec((2, C2 // 2), lambda t: (0, 0)),
        ],
        out_specs=pl.BlockSpec((RT, C2), lambda t: (t, 0)),
        out_shape=jax.ShapeDtypeStruct((R, C2), jnp.float32),
    )(m, st2, gb2)


def kernel(x, fixed_knn_graph, W1, g1, b1, W2, g2, b2):
    B, F, N = x.shape
    K = fixed_knn_graph.shape[-1]
    BN = B * N
    E = BN * K

    xt = jnp.transpose(x, (0, 2, 1)).reshape(BN, F)
    wnT = jnp.transpose(W1[:, :F])
    wcT = jnp.transpose(W1[:, F:])
    pt, ut = _tables(xt, wnT, wcT)

    idx = fixed_knn_graph.astype(jnp.int32)
    idx = idx + (jnp.arange(B, dtype=jnp.int32) * N)[:, None, None]
    idx_kmaj = jnp.transpose(idx, (2, 0, 1)).reshape(E // 128, 128)

    g = _sc_gather(pt, idx_kmaj)
    gp = g.reshape(K, BN // 2, 2 * F)
    utp = ut.reshape(BN // 2, 2 * F)

    st1 = _stats1(gp, utp)
    gb1 = jnp.stack([g1, b1])
    gb2 = jnp.stack([g2, b2])
    w2t = jnp.transpose(W2)
    zc = jnp.zeros_like(w2t)
    w2d = jnp.concatenate(
        [jnp.concatenate([w2t, zc], axis=1),
         jnp.concatenate([zc, w2t], axis=1)], axis=0)
    m, st2 = _main_pass(gp, utp, st1, gb1, w2d, E)
    y = _finalize(m, st2, gb2, E)
    return y.reshape(B, N, -1).transpose(0, 2, 1)
